# baseline (speedup 1.0000x reference)
import functools
import math

import jax
import jax.numpy as jnp
import numpy as np
from jax import lax
from jax.experimental import pallas as pl
from jax.experimental.pallas import tpu as pltpu
from jax.experimental.pallas import tpu_sc as plsc

D_MODEL = 128
L_SEQ = 50
BM = 256
N_TOK = 4096 * 50

_info = plsc.get_sparse_core_info()
NC, NS = _info.num_cores, _info.num_subcores
NW = NC * NS
B_PER_W = N_TOK // NW
CHUNK = 128
N_CHUNK = B_PER_W // CHUNK


def _pe_const(d_model: int, max_len: int) -> np.ndarray:
    pos = np.arange(0, max_len, dtype=np.float32)[:, None]
    div = np.exp(
        np.arange(0, d_model, 2, dtype=np.float32) * (-math.log(10000.0) / d_model)
    )
    pe = np.zeros((max_len, d_model), dtype=np.float32)
    pe[:, 0::2] = np.sin(pos * div)
    pe[:, 1::2] = np.cos(pos * div)
    return pe


_PE50 = _pe_const(D_MODEL, L_SEQ)


_sc_mesh = plsc.VectorSubcoreMesh(core_axis_name="c", subcore_axis_name="s")


@functools.partial(
    pl.kernel,
    mesh=_sc_mesh,
    out_type=jax.ShapeDtypeStruct((N_TOK, D_MODEL), jnp.float32),
    scratch_types=[
        pltpu.VMEM((B_PER_W,), jnp.int32),
        pltpu.VMEM((CHUNK, D_MODEL), jnp.float32),
        pltpu.SemaphoreType.DMA,
    ],
)
def _sc_gather(table_hbm, idx_hbm, out_hbm, idx_v, rows_v, sem):
    wid = lax.axis_index("s") * NC + lax.axis_index("c")
    base = wid * B_PER_W
    pltpu.sync_copy(idx_hbm.at[pl.ds(base, B_PER_W)], idx_v)

    def body(j, carry):
        pltpu.async_copy(
            table_hbm.at[idx_v.at[pl.ds(j * CHUNK, CHUNK)]], rows_v, sem
        ).wait()
        pltpu.sync_copy(rows_v, out_hbm.at[pl.ds(base + j * CHUNK, CHUNK)])
        return carry

    lax.fori_loop(0, N_CHUNK, body, 0)


def _tc_combine(xc_ref, emb_ref, wp_ref, peb_ref, out_ref):
    proj = jax.lax.dot_general(
        xc_ref[...], wp_ref[...], (((0,), (0,)), ((), ())),
        preferred_element_type=jnp.float32,
    )
    out_ref[...] = proj + emb_ref[...] + peb_ref[...]


def kernel(x_cont, x_month, x_weekday, x_dir, W_proj, b_proj, E_month, E_weekday, E_dir):
    B, L, C = x_cont.shape
    xc2 = x_cont.transpose(2, 1, 0)
    cidx = (
        x_month.astype(jnp.int32)
        + 12 * x_weekday.astype(jnp.int32)
        + 84 * x_dir.astype(jnp.int32)
    ).T.reshape(N_TOK)
    c = jnp.arange(672, dtype=jnp.int32)
    table = (
        jnp.take(E_month, c % 12, axis=0)
        + jnp.take(E_weekday, (c // 12) % 7, axis=0)
        + jnp.take(E_dir, c // 84, axis=0)
    )
    pe_block = jnp.asarray(_PE50)[:, None, :] + b_proj[None, None, :]

    emb = _sc_gather(table, cidx).reshape(L_SEQ, B, D_MODEL)

    out2 = pl.pallas_call(
        _tc_combine,
        grid=(B // BM,),
        in_specs=[
            pl.BlockSpec((C, L_SEQ, BM), lambda i: (0, 0, i)),
            pl.BlockSpec((L_SEQ, BM, D_MODEL), lambda i: (0, i, 0)),
            pl.BlockSpec((C, D_MODEL), lambda i: (0, 0)),
            pl.BlockSpec((L_SEQ, 1, D_MODEL), lambda i: (0, 0, 0)),
        ],
        out_specs=pl.BlockSpec((L_SEQ, BM, D_MODEL), lambda i: (0, i, 0)),
        out_shape=jax.ShapeDtypeStruct((L_SEQ, B, D_MODEL), jnp.float32),
        compiler_params=pltpu.CompilerParams(
            dimension_semantics=("parallel",),
        ),
    )(xc2, emb, W_proj, pe_block)
    return out2.transpose(1, 0, 2)


# speedup vs baseline: 5.1286x; 5.1286x over previous
import math

import jax
import jax.numpy as jnp
import numpy as np
from jax.experimental import pallas as pl
from jax.experimental.pallas import tpu as pltpu

D_MODEL = 128
L_SEQ = 50
BM = 512
R = BM * L_SEQ


def _pe_const(d_model: int, max_len: int) -> np.ndarray:
    pos = np.arange(0, max_len, dtype=np.float32)[:, None]
    div = np.exp(
        np.arange(0, d_model, 2, dtype=np.float32) * (-math.log(10000.0) / d_model)
    )
    pe = np.zeros((max_len, d_model), dtype=np.float32)
    pe[:, 0::2] = np.sin(pos * div)
    pe[:, 1::2] = np.cos(pos * div)
    return pe


_PE50 = _pe_const(D_MODEL, L_SEQ)


def _fused_kernel(xc_ref, combo_ref, wall_ref, peb_ref, out_ref):
    xct = xc_ref[...]
    combo = combo_ref[...][None]
    m = combo & 15
    w = (combo >> 4) & 15
    d = combo >> 8
    iota = jax.lax.broadcasted_iota(jnp.int32, (27, L_SEQ, BM), 0)
    oht = ((iota == m) | (iota == w + 12) | (iota == d + 19)).astype(jnp.float32)
    feat = jnp.concatenate([xct, oht], axis=0)
    acc = jax.lax.dot_general(
        feat, wall_ref[...], (((0,), (0,)), ((), ())),
        preferred_element_type=jnp.float32,
    )
    out_ref[...] = acc + peb_ref[...]


def kernel(x_cont, x_month, x_weekday, x_dir, W_proj, b_proj, E_month, E_weekday, E_dir):
    B, L, C = x_cont.shape
    xc2 = x_cont.transpose(2, 1, 0)
    combo = (
        x_month.astype(jnp.int32)
        | (x_weekday.astype(jnp.int32) << 4)
        | (x_dir.astype(jnp.int32) << 8)
    ).T
    w_all = jnp.concatenate([W_proj, E_month, E_weekday, E_dir], axis=0)
    pe_block = jnp.asarray(_PE50)[:, None, :] + b_proj[None, None, :]

    grid = (B // BM,)
    out2 = pl.pallas_call(
        _fused_kernel,
        grid=grid,
        in_specs=[
            pl.BlockSpec((C, L_SEQ, BM), lambda i: (0, 0, i)),
            pl.BlockSpec((L_SEQ, BM), lambda i: (0, i)),
            pl.BlockSpec((C + 27, D_MODEL), lambda i: (0, 0)),
            pl.BlockSpec((L_SEQ, 1, D_MODEL), lambda i: (0, 0, 0)),
        ],
        out_specs=pl.BlockSpec((L_SEQ, BM, D_MODEL), lambda i: (0, i, 0)),
        out_shape=jax.ShapeDtypeStruct((L, B, D_MODEL), jnp.float32),
        compiler_params=pltpu.CompilerParams(
            dimension_semantics=("parallel",),
        ),
    )(xc2, combo, w_all, pe_block)
    return out2.transpose(1, 0, 2)
